# baseline (speedup 1.0000x reference)
import functools

import jax
import jax.numpy as jnp
from jax import lax
from jax.experimental import pallas as pl
from jax.experimental.pallas import tpu as pltpu
from jax.experimental.pallas import tpu_sc as plsc


def kernel(x, pos_table):
    B, S, D = x.shape
    M = pos_table.shape[0]
    info = plsc.get_sparse_core_info()
    NC, NS, L = info.num_cores, info.num_subcores, info.num_lanes
    NW = NC * NS
    PPW = S // NW
    KPW = PPW // 2
    WIN = KPW + 16
    CH = 16
    NSUB = PPW // CH
    scale = (M - 1) / (S - 1)
    mesh = plsc.VectorSubcoreMesh(core_axis_name="c", subcore_axis_name="s")

    @functools.partial(
        pl.kernel,
        mesh=mesh,
        out_type=jax.ShapeDtypeStruct((B, S, D), jnp.float32),
        scratch_types=[
            pltpu.VMEM((WIN, D), jnp.float32),
            pltpu.VMEM((CH, D), jnp.float32),
            pltpu.VMEM((CH, D), jnp.float32),
        ],
    )
    def run(x_hbm, t_hbm, out_hbm, buf, emb, io):
        wid = lax.axis_index("s") * NC + lax.axis_index("c")
        k0 = wid * KPW
        pos0 = wid * PPW
        start = pl.multiple_of(jnp.clip(k0 - 8, 0, M - WIN), 8)
        pltpu.sync_copy(t_hbm.at[pl.ds(start, WIN)], buf)

        def sub_body(sub, carry):
            p0 = pl.multiple_of(pos0 + sub * CH, 8)

            def pos_body(p, carry2):
                i = p0 + p
                f = jnp.maximum((i - 1) // 2, 0)
                ra = f - start
                w = (jnp.full((L,), i, jnp.float32) * scale
                     - jnp.full((L,), f, jnp.float32))
                omw = 1.0 - w
                for d in range(D // L):
                    av = buf[ra, pl.ds(d * L, L)]
                    bv = buf[ra + 1, pl.ds(d * L, L)]
                    emb[p, pl.ds(d * L, L)] = omw * av + w * bv
                return carry2

            lax.fori_loop(0, CH, pos_body, 0)

            for b in range(B):
                pltpu.sync_copy(x_hbm.at[b, pl.ds(p0, CH)], io)

                def add_body(p, carry3):
                    for d in range(D // L):
                        plsc.addupdate(io.at[p, pl.ds(d * L, L)],
                                       emb[p, pl.ds(d * L, L)])
                    return carry3

                lax.fori_loop(0, CH, add_body, 0)
                pltpu.sync_copy(io, out_hbm.at[b, pl.ds(p0, CH)])
            return carry

        lax.fori_loop(0, NSUB, sub_body, 0)

    return run(x, pos_table)


# speedup vs baseline: 1.5869x; 1.5869x over previous
import functools

import jax
import jax.numpy as jnp
from jax import lax
from jax.experimental import pallas as pl
from jax.experimental.pallas import tpu as pltpu
from jax.experimental.pallas import tpu_sc as plsc


def kernel(x, pos_table):
    B, S, D = x.shape
    M = pos_table.shape[0]
    info = plsc.get_sparse_core_info()
    NC, NS, L = info.num_cores, info.num_subcores, info.num_lanes
    NW = NC * NS
    PPW = S // NW
    KPW = PPW // 2
    WIN = KPW + 9
    CH = 16
    NSUB = PPW // CH
    T = NSUB * B
    scale = (M - 1) / (S - 1)
    mesh = plsc.VectorSubcoreMesh(core_axis_name="c", subcore_axis_name="s")

    def mo8(v):
        return pl.multiple_of(v, 8)

    @functools.partial(
        pl.kernel,
        mesh=mesh,
        out_type=jax.ShapeDtypeStruct((B, S, D), jnp.float32),
        scratch_types=[
            pltpu.VMEM((WIN, D), jnp.float32),
            pltpu.VMEM((CH, D), jnp.float32),
            pltpu.VMEM((CH, D), jnp.float32),
            pltpu.VMEM((CH, D), jnp.float32),
            pltpu.SemaphoreType.DMA,
            pltpu.SemaphoreType.DMA,
            pltpu.SemaphoreType.DMA,
            pltpu.SemaphoreType.DMA,
        ],
    )
    def run(x_hbm, t_hbm, out_hbm, buf, emb, io0, io1, si0, si1, so0, so1):
        wid = lax.axis_index("s") * NC + lax.axis_index("c")
        k0 = wid * KPW
        pos0 = wid * PPW
        pltpu.sync_copy(t_hbm.at[pl.ds(mo8(jnp.maximum(k0 - 8, 0)), 8)],
                        buf.at[pl.ds(0, 8)])
        pltpu.sync_copy(t_hbm.at[pl.ds(mo8(k0), KPW)],
                        buf.at[pl.ds(8, KPW)])
        pltpu.sync_copy(t_hbm.at[pl.ds(mo8(jnp.minimum(k0 + KPW, M - 8)), 1)],
                        buf.at[pl.ds(KPW + 8, 1)])

        def step_coords(t):
            sub = t // 4
            b = t % 4
            p0 = mo8(pos0 + sub * CH)
            return b, p0

        def build_emb(sub):
            p0 = pos0 + sub * CH

            @plsc.parallel_loop(0, CH, 1, unroll=2)
            def _pos(p):
                i = p0 + p
                f = jnp.maximum((i - 1) // 2, 0)
                ra = f - k0 + 8
                w = (jnp.full((L,), i, jnp.float32) * scale
                     - jnp.full((L,), f, jnp.float32))
                omw = 1.0 - w
                for d in range(D // L):
                    av = buf[ra, pl.ds(d * L, L)]
                    bv = buf[ra + 1, pl.ds(d * L, L)]
                    emb[p, pl.ds(d * L, L)] = omw * av + w * bv

        def add_emb(io):
            @plsc.parallel_loop(0, CH, 1, unroll=2)
            def _add(p):
                for d in range(D // L):
                    plsc.addupdate(io.at[p, pl.ds(d * L, L)],
                                   emb[p, pl.ds(d * L, L)])

        b0, q0 = step_coords(0)
        pltpu.async_copy(x_hbm.at[b0, pl.ds(q0, CH)], io0, si0)

        def g_body(g, carry):
            t0 = 2 * g
            t1 = 2 * g + 1
            bA, pA = step_coords(t0)
            bB, pB = step_coords(t1)

            @pl.when(g % 2 == 0)
            def _():
                build_emb(t0 // 4)

            pltpu.make_async_copy(x_hbm.at[bA, pl.ds(pA, CH)], io0, si0).wait()
            add_emb(io0)
            pltpu.async_copy(io0, out_hbm.at[bA, pl.ds(pA, CH)], so0)

            @pl.when(g > 0)
            def _():
                pltpu.make_async_copy(io1, out_hbm.at[0, pl.ds(0, CH)],
                                      so1).wait()
            bN, pN = step_coords(t1)
            pltpu.async_copy(x_hbm.at[bN, pl.ds(pN, CH)], io1, si1)

            pltpu.make_async_copy(x_hbm.at[bB, pl.ds(pB, CH)], io1, si1).wait()
            add_emb(io1)
            pltpu.async_copy(io1, out_hbm.at[bB, pl.ds(pB, CH)], so1)

            @pl.when(g < T // 2 - 1)
            def _():
                pltpu.make_async_copy(io0, out_hbm.at[0, pl.ds(0, CH)],
                                      so0).wait()
                bM, pM = step_coords(t0 + 2)
                pltpu.async_copy(x_hbm.at[bM, pl.ds(pM, CH)], io0, si0)

            return carry

        lax.fori_loop(0, T // 2, g_body, 0)
        pltpu.make_async_copy(io0, out_hbm.at[0, pl.ds(0, CH)], so0).wait()
        pltpu.make_async_copy(io1, out_hbm.at[0, pl.ds(0, CH)], so1).wait()

    return run(x, pos_table)
